# Initial kernel scaffold; baseline (speedup 1.0000x reference)
import jax
import jax.numpy as jnp
from jax.experimental import pallas as pl


def kernel(x, Wc1, b1, g1, be1, rm1, rv1, Wc2, g2, be2, rm2, rv2):
    raise NotImplementedError("write your pallas kernel here")


# speedup vs baseline: 3.1072x; 3.1072x over previous
import functools

import jax
import jax.numpy as jnp
import numpy as np
from jax import lax
from jax.experimental import pallas as pl
from jax.experimental.pallas import tpu as pltpu
from jax.experimental.pallas import tpu_sc as plsc

KW = 5
PAD = 2
NUM = 32
H2 = W2 = 40
HW = H2 * W2
PW = H2 + 2 * PAD
PSZ = PW * PW
HI = lax.Precision.HIGHEST


def _resize_wmat(n):
    m = np.zeros((n, 2 * n), np.float32)
    for o in range(2 * n):
        c = (o + 0.5) / 2 - 0.5
        k0 = int(np.floor(c))
        w1 = c - k0
        tot = sum(w for k, w in ((k0, 1 - w1), (k0 + 1, w1)) if 0 <= k < n)
        for k, w in ((k0, 1 - w1), (k0 + 1, w1)):
            if 0 <= k < n:
                m[k, o] = w / tot
    return m


_M20T_NP = np.ascontiguousarray(_resize_wmat(20).T)


def _dense_body(x_ref, m_ref, w1_ref, w2_ref, v1_ref, v2_ref,
                xfp_ref, msk_ref):
    xb = x_ref[0].reshape(20, 20 * 512)
    s1 = jnp.dot(m_ref[...], xb, precision=HI)
    s1 = s1.reshape(40, 20, 512)
    t1 = jnp.transpose(s1, (1, 0, 2)).reshape(20, 40 * 512)
    s2 = jnp.dot(m_ref[...], t1, precision=HI)
    u = jnp.transpose(s2.reshape(40, 40, 512), (1, 0, 2))

    up = jnp.pad(u, ((1, 1), (1, 1), (0, 0)))
    acc = None
    for t in range(9):
        i, j = t // 3, t % 3
        xt = up[i:i + 40, j:j + 40, :].reshape(HW, 512).astype(jnp.bfloat16)
        p = jnp.dot(xt, w1_ref[t], preferred_element_type=jnp.float32)
        acc = p if acc is None else acc + p

    b1 = v1_ref[0:1, :]
    rm1 = v1_ref[1:2, :]
    d1 = v1_ref[2:3, :]
    g1 = v1_ref[3:4, :]
    be1 = v1_ref[4:5, :]
    y = acc + b1
    y = (y - rm1) / d1 * g1 + be1
    z = jnp.maximum(y, 0.0)

    y2 = jnp.dot(z.astype(jnp.bfloat16), w2_ref[...],
                 preferred_element_type=jnp.float32)
    rm2 = v2_ref[0:1, :]
    d2 = v2_ref[1:2, :]
    g2 = v2_ref[2:3, :]
    be2 = v2_ref[3:4, :]
    y2 = (y2 - rm2) / d2 * g2 + be2

    xf = jnp.where(y2 > 1.0, y2, 0.0)
    xf3 = xf.reshape(40, 40, 32)
    xp3 = jnp.pad(xf3, ((PAD, PAD), (PAD, PAD), (0, 0)))
    xfp_ref[0] = xp3.reshape(PSZ, 32)

    ws = None
    for t in range(25):
        i, j = t // 5, t % 5
        sl = xp3[i:i + 40, j:j + 40, :]
        ws = sl if ws is None else ws + sl

    wm = jnp.pad(ws, ((PAD, PAD), (PAD, PAD), (0, 0)),
                 constant_values=-jnp.inf)
    mp = None
    for t in range(25):
        i, j = t // 5, t % 5
        sl = wm[i:i + 40, j:j + 40, :]
        mp = sl if mp is None else jnp.maximum(mp, sl)
    msk_ref[0] = jnp.where(ws == mp, ws, 0.0).reshape(HW, 32)


def _dense_stage(xT, m20t, w1b, w2b, v1, v2):
    return pl.pallas_call(
        _dense_body,
        grid=(4,),
        in_specs=[
            pl.BlockSpec((1, 20, 20, 512), lambda b: (b, 0, 0, 0)),
            pl.BlockSpec((40, 20), lambda b: (0, 0)),
            pl.BlockSpec((9, 512, 512), lambda b: (0, 0, 0)),
            pl.BlockSpec((512, 32), lambda b: (0, 0)),
            pl.BlockSpec((5, 512), lambda b: (0, 0)),
            pl.BlockSpec((4, 32), lambda b: (0, 0)),
        ],
        out_specs=[
            pl.BlockSpec((1, PSZ, 32), lambda b: (b, 0, 0)),
            pl.BlockSpec((1, HW, 32), lambda b: (b, 0, 0)),
        ],
        out_shape=[
            jax.ShapeDtypeStruct((4, PSZ, 32), jnp.float32),
            jax.ShapeDtypeStruct((4, HW, 32), jnp.float32),
        ],
    )(xT, m20t, w1b, w2b, v1, v2)


def _topk_gather_stage(masked_t, xfpad_t):
    info = plsc.get_sparse_core_info()
    nc, ns = info.num_cores, info.num_subcores
    nw = nc * ns
    tasks_per_w = 128 // nw
    mesh = plsc.VectorSubcoreMesh(core_axis_name="c", subcore_axis_name="s")

    mflat = masked_t.reshape(128 * HW)
    xflat = xfpad_t.reshape(128 * PSZ)

    @functools.partial(
        pl.kernel,
        mesh=mesh,
        out_type=[
            jax.ShapeDtypeStruct((128 * NUM * 25,), jnp.float32),
            jax.ShapeDtypeStruct((128 * NUM * 4,), jnp.int32),
        ],
        scratch_types=[
            pltpu.VMEM((HW,), jnp.float32),
            pltpu.VMEM((PSZ,), jnp.float32),
            pltpu.VMEM((NUM * 25,), jnp.float32),
            pltpu.VMEM((NUM * 4,), jnp.int32),
        ],
        compiler_params=pltpu.CompilerParams(needs_layout_passes=False),
    )
    def k(m_hbm, xf_hbm, tw_hbm, pos_hbm, vals_v, xfp_v, tw_v, pos_v):
        wid = lax.axis_index("s") * nc + lax.axis_index("c")
        lane = lax.iota(jnp.int32, 16)
        offs0 = (lane // 5) * PW + lane % 5
        l16 = lane + 16
        offs1 = jnp.minimum((l16 // 5) * PW + l16 % 5, PSZ - 1)
        m1 = lane < 9
        m_one = lane < 1
        m_four = lane < 4
        neg1 = jnp.full((16,), -1.0, jnp.float32)

        _dn = lax.GatherDimensionNumbers(offset_dims=(),
                                         collapsed_slice_dims=(0,),
                                         start_index_map=(0,))

        def lane_perm(x, idx):
            return lax.gather(x, idx[:, None], _dn, (1,),
                              mode=lax.GatherScatterMode.PROMISE_IN_BOUNDS)

        def task_body(tj, _):
            task = wid * tasks_per_w + tj
            pltpu.sync_copy(m_hbm.at[pl.ds(task * HW, HW)], vals_v)
            pltpu.sync_copy(xf_hbm.at[pl.ds(task * PSZ, PSZ)], xfp_v)

            winners = []
            for _n in range(NUM):
                def scan_body(kk, carry):
                    bv, bp = carry
                    v = vals_v[pl.ds(kk * 16, 16)]
                    p = kk * 16 + lane
                    upd = v > bv
                    return (jnp.where(upd, v, bv), jnp.where(upd, p, bp))
                bv, bp = lax.fori_loop(
                    0, HW // 16, scan_body,
                    (jnp.full((16,), -1.0, jnp.float32),
                     jnp.full((16,), 99999, jnp.int32)))
                mx = bv
                for s in (1, 2, 4, 8):
                    mx = jnp.maximum(mx, lane_perm(mx, lane ^ s))
                pidx = jnp.where(bv == mx, bp, 99999)
                w = pidx
                for s in (1, 2, 4, 8):
                    w = jnp.minimum(w, lane_perm(w, lane ^ s))
                winners.append(w)
                plsc.store_scatter(vals_v, [w], neg1, mask=m_one)

            for n, w in enumerate(winners):
                hh = w // W2
                ww = w % W2
                base = hh * PW + ww
                g0 = plsc.load_gather(xfp_v, [base + offs0])
                tw_v[pl.ds(n * 25, 16)] = g0
                g1 = plsc.load_gather(xfp_v, [base + offs1], mask=m1)
                plsc.store_scatter(tw_v, [n * 25 + 16 + lane], g1, mask=m1)
                x1 = jnp.clip(ww - PAD, 0, W2 - 1)
                y1 = jnp.clip(hh - PAD, 0, H2 - 1)
                x2 = jnp.clip(ww + PAD, 0, W2 - 1)
                y2 = jnp.clip(hh + PAD, 0, H2 - 1)
                pv = jnp.where(lane == 0, x1,
                               jnp.where(lane == 1, y1,
                                         jnp.where(lane == 2, x2, y2)))
                plsc.store_scatter(pos_v, [4 * n + lane], pv, mask=m_four)

            pltpu.sync_copy(tw_v, tw_hbm.at[pl.ds(task * NUM * 25, NUM * 25)])
            pltpu.sync_copy(pos_v, pos_hbm.at[pl.ds(task * NUM * 4, NUM * 4)])
            return 0

        lax.fori_loop(0, tasks_per_w, task_body, 0)

    return k(mflat, xflat)


def kernel(x, Wc1, b1, g1, be1, rm1, rv1, Wc2, g2, be2, rm2, rv2):
    xT = jnp.transpose(x, (0, 2, 3, 1))
    w1b = jnp.transpose(Wc1, (2, 3, 1, 0)).reshape(9, 512, 512).astype(jnp.bfloat16)
    w2b = Wc2[:, :, 0, 0].T.astype(jnp.bfloat16)
    d1 = jnp.sqrt(rv1 + 1e-5)
    d2 = jnp.sqrt(rv2 + 1e-5)
    v1 = jnp.stack([b1, rm1, d1, g1, be1])
    v2 = jnp.stack([rm2, d2, g2, be2])

    xfp, msk = _dense_stage(xT, jnp.asarray(_M20T_NP), w1b, w2b, v1, v2)
    masked_t = jnp.transpose(msk, (0, 2, 1))
    xfpad_t = jnp.transpose(xfp, (0, 2, 1))

    tw_f, pos_f = _topk_gather_stage(masked_t, xfpad_t)
    tw = tw_f.reshape(4, 32, NUM, KW, KW)
    pos = pos_f.reshape(4, 32, NUM, 4)
    return tw, pos


# speedup vs baseline: 3.5984x; 1.1581x over previous
import functools

import jax
import jax.numpy as jnp
import numpy as np
from jax import lax
from jax.experimental import pallas as pl
from jax.experimental.pallas import tpu as pltpu
from jax.experimental.pallas import tpu_sc as plsc

KW = 5
PAD = 2
NUM = 32
H2 = W2 = 40
HW = H2 * W2
PW = H2 + 2 * PAD
PSZ = PW * PW
HI = lax.Precision.HIGHEST


def _resize_wmat(n):
    m = np.zeros((n, 2 * n), np.float32)
    for o in range(2 * n):
        c = (o + 0.5) / 2 - 0.5
        k0 = int(np.floor(c))
        w1 = c - k0
        tot = sum(w for k, w in ((k0, 1 - w1), (k0 + 1, w1)) if 0 <= k < n)
        for k, w in ((k0, 1 - w1), (k0 + 1, w1)):
            if 0 <= k < n:
                m[k, o] = w / tot
    return m


_M20T_NP = np.ascontiguousarray(_resize_wmat(20).T)


def _dense_body(x_ref, m_ref, w1_ref, w2_ref, v1_ref, v2_ref,
                xfp_ref, msk_ref):
    xb = x_ref[0].reshape(20, 20 * 512)
    s1 = jnp.dot(m_ref[...], xb, precision=HI)
    s1 = s1.reshape(40, 20, 512)
    t1 = jnp.transpose(s1, (1, 0, 2)).reshape(20, 40 * 512)
    s2 = jnp.dot(m_ref[...], t1, precision=HI)
    u = jnp.transpose(s2.reshape(40, 40, 512), (1, 0, 2))

    up = jnp.pad(u, ((1, 1), (1, 1), (0, 0)))
    acc = None
    for t in range(9):
        i, j = t // 3, t % 3
        xt = up[i:i + 40, j:j + 40, :].reshape(HW, 512).astype(jnp.bfloat16)
        p = jnp.dot(xt, w1_ref[t], preferred_element_type=jnp.float32)
        acc = p if acc is None else acc + p

    b1 = v1_ref[0:1, :]
    rm1 = v1_ref[1:2, :]
    d1 = v1_ref[2:3, :]
    g1 = v1_ref[3:4, :]
    be1 = v1_ref[4:5, :]
    y = acc + b1
    y = (y - rm1) / d1 * g1 + be1
    z = jnp.maximum(y, 0.0)

    y2 = jnp.dot(z.astype(jnp.bfloat16), w2_ref[...],
                 preferred_element_type=jnp.float32)
    rm2 = v2_ref[0:1, :]
    d2 = v2_ref[1:2, :]
    g2 = v2_ref[2:3, :]
    be2 = v2_ref[3:4, :]
    y2 = (y2 - rm2) / d2 * g2 + be2

    xf = jnp.where(y2 > 1.0, y2, 0.0)
    xf3 = xf.reshape(40, 40, 32)
    xp3 = jnp.pad(xf3, ((PAD, PAD), (PAD, PAD), (0, 0)))
    xfp_ref[0] = xp3.reshape(PSZ, 32)

    ws = None
    for t in range(25):
        i, j = t // 5, t % 5
        sl = xp3[i:i + 40, j:j + 40, :]
        ws = sl if ws is None else ws + sl

    wm = jnp.pad(ws, ((PAD, PAD), (PAD, PAD), (0, 0)),
                 constant_values=-jnp.inf)
    mp = None
    for t in range(25):
        i, j = t // 5, t % 5
        sl = wm[i:i + 40, j:j + 40, :]
        mp = sl if mp is None else jnp.maximum(mp, sl)
    msk_ref[0] = jnp.where(ws == mp, ws, 0.0).reshape(HW, 32)


def _dense_stage(xT, m20t, w1b, w2b, v1, v2):
    return pl.pallas_call(
        _dense_body,
        grid=(4,),
        in_specs=[
            pl.BlockSpec((1, 20, 20, 512), lambda b: (b, 0, 0, 0)),
            pl.BlockSpec((40, 20), lambda b: (0, 0)),
            pl.BlockSpec((9, 512, 512), lambda b: (0, 0, 0)),
            pl.BlockSpec((512, 32), lambda b: (0, 0)),
            pl.BlockSpec((5, 512), lambda b: (0, 0)),
            pl.BlockSpec((4, 32), lambda b: (0, 0)),
        ],
        out_specs=[
            pl.BlockSpec((1, PSZ, 32), lambda b: (b, 0, 0)),
            pl.BlockSpec((1, HW, 32), lambda b: (b, 0, 0)),
        ],
        out_shape=[
            jax.ShapeDtypeStruct((4, PSZ, 32), jnp.float32),
            jax.ShapeDtypeStruct((4, HW, 32), jnp.float32),
        ],
    )(xT, m20t, w1b, w2b, v1, v2)


def _topk_gather_stage(masked_t, xfpad_t):
    info = plsc.get_sparse_core_info()
    nc, ns = info.num_cores, info.num_subcores
    nw = nc * ns
    tasks_per_w = 128 // nw
    mesh = plsc.VectorSubcoreMesh(core_axis_name="c", subcore_axis_name="s")

    mflat = masked_t.reshape(128 * HW)
    xflat = xfpad_t.reshape(128 * PSZ)

    @functools.partial(
        pl.kernel,
        mesh=mesh,
        out_type=[
            jax.ShapeDtypeStruct((128 * NUM * 25,), jnp.float32),
            jax.ShapeDtypeStruct((128 * NUM * 4,), jnp.int32),
        ],
        scratch_types=[
            pltpu.VMEM((HW,), jnp.float32),
            pltpu.VMEM((PSZ,), jnp.float32),
            pltpu.VMEM((NUM * 25,), jnp.float32),
            pltpu.VMEM((NUM * 4,), jnp.int32),
            pltpu.VMEM((112,), jnp.float32),
            pltpu.VMEM((112,), jnp.int32),
        ],
        compiler_params=pltpu.CompilerParams(needs_layout_passes=False),
    )
    def k(m_hbm, xf_hbm, tw_hbm, pos_hbm, vals_v, xfp_v, tw_v, pos_v, sv_v, sp_v):
        wid = lax.axis_index("s") * nc + lax.axis_index("c")
        lane = lax.iota(jnp.int32, 16)
        offs0 = (lane // 5) * PW + lane % 5
        l16 = lane + 16
        offs1 = jnp.minimum((l16 // 5) * PW + l16 % 5, PSZ - 1)
        m1 = lane < 9
        m_one = lane < 1
        m_four = lane < 4
        neg1 = jnp.full((16,), -1.0, jnp.float32)

        _dn = lax.GatherDimensionNumbers(offset_dims=(),
                                         collapsed_slice_dims=(0,),
                                         start_index_map=(0,))

        def lane_perm(x, idx):
            return lax.gather(x, idx[:, None], _dn, (1,),
                              mode=lax.GatherScatterMode.PROMISE_IN_BOUNDS)

        def bmax(x):
            for s in (1, 2, 4, 8):
                x = jnp.maximum(x, lane_perm(x, lane ^ s))
            return x

        def bmin(x):
            for s in (1, 2, 4, 8):
                x = jnp.minimum(x, lane_perm(x, lane ^ s))
            return x

        def task_body(tj, _):
            task = wid * tasks_per_w + tj
            pltpu.sync_copy(m_hbm.at[pl.ds(task * HW, HW)], vals_v)
            pltpu.sync_copy(xf_hbm.at[pl.ds(task * PSZ, PSZ)], xfp_v)

            big = jnp.full((16,), 99999, jnp.int32)
            for j in range(7):
                sv_v[pl.ds(j * 16, 16)] = neg1
                sp_v[pl.ds(j * 16, 16)] = big

            def build_body(kk, _c):
                v = vals_v[pl.ds(kk * 16, 16)]
                mxv = bmax(v)
                mpv = bmin(jnp.where(v == mxv, kk * 16 + lane, 99999))
                kvec = jnp.zeros((16,), jnp.int32) + kk
                plsc.store_scatter(sv_v, [kvec], mxv, mask=m_one)
                plsc.store_scatter(sp_v, [kvec], mpv, mask=m_one)
                return 0
            lax.fori_loop(0, HW // 16, build_body, 0)

            winners = []
            for _n in range(NUM):
                bv = jnp.full((16,), -2.0, jnp.float32)
                bp = jnp.full((16,), 99999, jnp.int32)
                for j in range(7):
                    v = sv_v[pl.ds(j * 16, 16)]
                    p = sp_v[pl.ds(j * 16, 16)]
                    upd = v > bv
                    bv = jnp.where(upd, v, bv)
                    bp = jnp.where(upd, p, bp)
                mx = bmax(bv)
                w = bmin(jnp.where(bv == mx, bp, 99999))
                winners.append(w)
                kb = w // 16
                bidx = kb * 16 + lane
                v = plsc.load_gather(vals_v, [bidx])
                v = jnp.where(bidx == w, -1.0, v)
                plsc.store_scatter(vals_v, [w], neg1, mask=m_one)
                m2 = bmax(v)
                mp2 = bmin(jnp.where(v == m2, bidx, 99999))
                plsc.store_scatter(sv_v, [kb], m2, mask=m_one)
                plsc.store_scatter(sp_v, [kb], mp2, mask=m_one)

            for n, w in enumerate(winners):
                hh = w // W2
                ww = w % W2
                base = hh * PW + ww
                g0 = plsc.load_gather(xfp_v, [base + offs0])
                tw_v[pl.ds(n * 25, 16)] = g0
                g1 = plsc.load_gather(xfp_v, [base + offs1], mask=m1)
                plsc.store_scatter(tw_v, [n * 25 + 16 + lane], g1, mask=m1)
                x1 = jnp.clip(ww - PAD, 0, W2 - 1)
                y1 = jnp.clip(hh - PAD, 0, H2 - 1)
                x2 = jnp.clip(ww + PAD, 0, W2 - 1)
                y2 = jnp.clip(hh + PAD, 0, H2 - 1)
                pv = jnp.where(lane == 0, x1,
                               jnp.where(lane == 1, y1,
                                         jnp.where(lane == 2, x2, y2)))
                plsc.store_scatter(pos_v, [4 * n + lane], pv, mask=m_four)

            pltpu.sync_copy(tw_v, tw_hbm.at[pl.ds(task * NUM * 25, NUM * 25)])
            pltpu.sync_copy(pos_v, pos_hbm.at[pl.ds(task * NUM * 4, NUM * 4)])
            return 0

        lax.fori_loop(0, tasks_per_w, task_body, 0)

    return k(mflat, xflat)


def kernel(x, Wc1, b1, g1, be1, rm1, rv1, Wc2, g2, be2, rm2, rv2):
    xT = jnp.transpose(x, (0, 2, 3, 1))
    w1b = jnp.transpose(Wc1, (2, 3, 1, 0)).reshape(9, 512, 512).astype(jnp.bfloat16)
    w2b = Wc2[:, :, 0, 0].T.astype(jnp.bfloat16)
    d1 = jnp.sqrt(rv1 + 1e-5)
    d2 = jnp.sqrt(rv2 + 1e-5)
    v1 = jnp.stack([b1, rm1, d1, g1, be1])
    v2 = jnp.stack([rm2, d2, g2, be2])

    xfp, msk = _dense_stage(xT, jnp.asarray(_M20T_NP), w1b, w2b, v1, v2)
    masked_t = jnp.transpose(msk, (0, 2, 1))
    xfpad_t = jnp.transpose(xfp, (0, 2, 1))

    tw_f, pos_f = _topk_gather_stage(masked_t, xfpad_t)
    tw = tw_f.reshape(4, 32, NUM, KW, KW)
    pos = pos_f.reshape(4, 32, NUM, 4)
    return tw, pos


# speedup vs baseline: 3.7449x; 1.0407x over previous
import functools

import jax
import jax.numpy as jnp
import numpy as np
from jax import lax
from jax.experimental import pallas as pl
from jax.experimental.pallas import tpu as pltpu
from jax.experimental.pallas import tpu_sc as plsc

KW = 5
PAD = 2
NUM = 32
H2 = W2 = 40
HW = H2 * W2
PW = H2 + 2 * PAD
PSZ = PW * PW
HI = lax.Precision.HIGHEST


def _resize_wmat(n):
    m = np.zeros((n, 2 * n), np.float32)
    for o in range(2 * n):
        c = (o + 0.5) / 2 - 0.5
        k0 = int(np.floor(c))
        w1 = c - k0
        tot = sum(w for k, w in ((k0, 1 - w1), (k0 + 1, w1)) if 0 <= k < n)
        for k, w in ((k0, 1 - w1), (k0 + 1, w1)):
            if 0 <= k < n:
                m[k, o] = w / tot
    return m


_M20T_NP = np.ascontiguousarray(_resize_wmat(20).T)


def _dense_body(x_ref, m_ref, w1_ref, w2_ref, v1_ref, v2_ref,
                xfp_ref, msk_ref):
    xb = x_ref[0].reshape(20, 20 * 512)
    s1 = jnp.dot(m_ref[...], xb, precision=HI)
    s1 = s1.reshape(40, 20, 512)
    t1 = jnp.transpose(s1, (1, 0, 2)).reshape(20, 40 * 512)
    s2 = jnp.dot(m_ref[...], t1, precision=HI)
    u = jnp.transpose(s2.reshape(40, 40, 512), (1, 0, 2))

    up = jnp.pad(u, ((1, 1), (1, 1), (0, 0)))
    acc = None
    for t in range(9):
        i, j = t // 3, t % 3
        xt = up[i:i + 40, j:j + 40, :].reshape(HW, 512).astype(jnp.bfloat16)
        p = jnp.dot(xt, w1_ref[t], preferred_element_type=jnp.float32)
        acc = p if acc is None else acc + p

    b1 = v1_ref[0:1, :]
    rm1 = v1_ref[1:2, :]
    d1 = v1_ref[2:3, :]
    g1 = v1_ref[3:4, :]
    be1 = v1_ref[4:5, :]
    y = acc + b1
    y = (y - rm1) / d1 * g1 + be1
    z = jnp.maximum(y, 0.0)

    y2 = jnp.dot(z.astype(jnp.bfloat16), w2_ref[...],
                 preferred_element_type=jnp.float32)
    rm2 = v2_ref[0:1, :]
    d2 = v2_ref[1:2, :]
    g2 = v2_ref[2:3, :]
    be2 = v2_ref[3:4, :]
    y2 = (y2 - rm2) / d2 * g2 + be2

    xf = jnp.where(y2 > 1.0, y2, 0.0)
    xf3 = xf.reshape(40, 40, 32)
    xp3 = jnp.pad(xf3, ((PAD, PAD), (PAD, PAD), (0, 0)))
    xfp_ref[0] = xp3.reshape(PSZ, 32)

    ws = None
    for t in range(25):
        i, j = t // 5, t % 5
        sl = xp3[i:i + 40, j:j + 40, :]
        ws = sl if ws is None else ws + sl

    wm = jnp.pad(ws, ((PAD, PAD), (PAD, PAD), (0, 0)),
                 constant_values=-jnp.inf)
    mp = None
    for t in range(25):
        i, j = t // 5, t % 5
        sl = wm[i:i + 40, j:j + 40, :]
        mp = sl if mp is None else jnp.maximum(mp, sl)
    msk_ref[0] = jnp.where(ws == mp, ws, 0.0).reshape(HW, 32)


def _dense_stage(xT, m20t, w1b, w2b, v1, v2):
    return pl.pallas_call(
        _dense_body,
        grid=(4,),
        in_specs=[
            pl.BlockSpec((1, 20, 20, 512), lambda b: (b, 0, 0, 0)),
            pl.BlockSpec((40, 20), lambda b: (0, 0)),
            pl.BlockSpec((9, 512, 512), lambda b: (0, 0, 0)),
            pl.BlockSpec((512, 32), lambda b: (0, 0)),
            pl.BlockSpec((5, 512), lambda b: (0, 0)),
            pl.BlockSpec((4, 32), lambda b: (0, 0)),
        ],
        out_specs=[
            pl.BlockSpec((1, PSZ, 32), lambda b: (b, 0, 0)),
            pl.BlockSpec((1, HW, 32), lambda b: (b, 0, 0)),
        ],
        out_shape=[
            jax.ShapeDtypeStruct((4, PSZ, 32), jnp.float32),
            jax.ShapeDtypeStruct((4, HW, 32), jnp.float32),
        ],
    )(xT, m20t, w1b, w2b, v1, v2)


def _topk_gather_stage(masked_t, xfpad_t):
    info = plsc.get_sparse_core_info()
    nc, ns = info.num_cores, info.num_subcores
    nw = nc * ns
    tasks_per_w = 128 // nw
    mesh = plsc.VectorSubcoreMesh(core_axis_name="c", subcore_axis_name="s")

    mflat = masked_t.reshape(128 * HW)
    xflat = xfpad_t.reshape(128 * PSZ)

    @functools.partial(
        pl.kernel,
        mesh=mesh,
        out_type=[
            jax.ShapeDtypeStruct((128 * NUM * 25,), jnp.float32),
            jax.ShapeDtypeStruct((128 * NUM * 4,), jnp.int32),
        ],
        scratch_types=[
            pltpu.VMEM((HW,), jnp.float32),
            pltpu.VMEM((PSZ,), jnp.float32),
            pltpu.VMEM((NUM * 25,), jnp.float32),
            pltpu.VMEM((NUM * 4,), jnp.int32),
            pltpu.VMEM((112,), jnp.float32),
        ],
        compiler_params=pltpu.CompilerParams(needs_layout_passes=False),
    )
    def k(m_hbm, xf_hbm, tw_hbm, pos_hbm, vals_v, xfp_v, tw_v, pos_v, sv_v):
        wid = lax.axis_index("s") * nc + lax.axis_index("c")
        lane = lax.iota(jnp.int32, 16)
        offs0 = (lane // 5) * PW + lane % 5
        l16 = lane + 16
        offs1 = jnp.minimum((l16 // 5) * PW + l16 % 5, PSZ - 1)
        m1 = lane < 9
        m_one = lane < 1
        m_four = lane < 4
        neg1 = jnp.full((16,), -1.0, jnp.float32)

        _dn = lax.GatherDimensionNumbers(offset_dims=(),
                                         collapsed_slice_dims=(0,),
                                         start_index_map=(0,))

        def lane_perm(x, idx):
            return lax.gather(x, idx[:, None], _dn, (1,),
                              mode=lax.GatherScatterMode.PROMISE_IN_BOUNDS)

        def bmax(x):
            for s in (1, 2, 4, 8):
                x = jnp.maximum(x, lane_perm(x, lane ^ s))
            return x

        def bmin(x):
            for s in (1, 2, 4, 8):
                x = jnp.minimum(x, lane_perm(x, lane ^ s))
            return x

        def task_body(tj, _):
            task = wid * tasks_per_w + tj
            pltpu.sync_copy(m_hbm.at[pl.ds(task * HW, HW)], vals_v)
            pltpu.sync_copy(xf_hbm.at[pl.ds(task * PSZ, PSZ)], xfp_v)

            for j in range(7):
                sv_v[pl.ds(j * 16, 16)] = neg1

            def build_body(kk, _c):
                for q in range(4):
                    b = kk * 4 + q
                    v = vals_v[pl.ds(b * 16, 16)]
                    mxv = bmax(v)
                    kvec = jnp.zeros((16,), jnp.int32) + b
                    plsc.store_scatter(sv_v, [kvec], mxv, mask=m_one)
                return 0
            lax.fori_loop(0, HW // 64, build_body, 0)

            winners = []
            for _n in range(NUM):
                bv = jnp.full((16,), -2.0, jnp.float32)
                bb = jnp.full((16,), 99999, jnp.int32)
                for j in range(7):
                    v = sv_v[pl.ds(j * 16, 16)]
                    bidv = j * 16 + lane
                    upd = v > bv
                    bv = jnp.where(upd, v, bv)
                    bb = jnp.where(upd, bidv, bb)
                mx = bmax(bv)
                kb = bmin(jnp.where(bv == mx, bb, 99999))
                bidx = kb * 16 + lane
                v = plsc.load_gather(vals_v, [bidx])
                w = bmin(jnp.where(v == mx, bidx, 99999))
                winners.append(w)
                v = jnp.where(bidx == w, -1.0, v)
                plsc.store_scatter(vals_v, [w], neg1, mask=m_one)
                m2 = bmax(v)
                plsc.store_scatter(sv_v, [kb], m2, mask=m_one)

            for n, w in enumerate(winners):
                hh = w // W2
                ww = w % W2
                base = hh * PW + ww
                g0 = plsc.load_gather(xfp_v, [base + offs0])
                tw_v[pl.ds(n * 25, 16)] = g0
                g1 = plsc.load_gather(xfp_v, [base + offs1], mask=m1)
                plsc.store_scatter(tw_v, [n * 25 + 16 + lane], g1, mask=m1)
                x1 = jnp.clip(ww - PAD, 0, W2 - 1)
                y1 = jnp.clip(hh - PAD, 0, H2 - 1)
                x2 = jnp.clip(ww + PAD, 0, W2 - 1)
                y2 = jnp.clip(hh + PAD, 0, H2 - 1)
                pv = jnp.where(lane == 0, x1,
                               jnp.where(lane == 1, y1,
                                         jnp.where(lane == 2, x2, y2)))
                plsc.store_scatter(pos_v, [4 * n + lane], pv, mask=m_four)

            pltpu.sync_copy(tw_v, tw_hbm.at[pl.ds(task * NUM * 25, NUM * 25)])
            pltpu.sync_copy(pos_v, pos_hbm.at[pl.ds(task * NUM * 4, NUM * 4)])
            return 0

        lax.fori_loop(0, tasks_per_w, task_body, 0)

    return k(mflat, xflat)


def kernel(x, Wc1, b1, g1, be1, rm1, rv1, Wc2, g2, be2, rm2, rv2):
    xT = jnp.transpose(x, (0, 2, 3, 1))
    w1b = jnp.transpose(Wc1, (2, 3, 1, 0)).reshape(9, 512, 512).astype(jnp.bfloat16)
    w2b = Wc2[:, :, 0, 0].T.astype(jnp.bfloat16)
    d1 = jnp.sqrt(rv1 + 1e-5)
    d2 = jnp.sqrt(rv2 + 1e-5)
    v1 = jnp.stack([b1, rm1, d1, g1, be1])
    v2 = jnp.stack([rm2, d2, g2, be2])

    xfp, msk = _dense_stage(xT, jnp.asarray(_M20T_NP), w1b, w2b, v1, v2)
    masked_t = jnp.transpose(msk, (0, 2, 1))
    xfpad_t = jnp.transpose(xfp, (0, 2, 1))

    tw_f, pos_f = _topk_gather_stage(masked_t, xfpad_t)
    tw = tw_f.reshape(4, 32, NUM, KW, KW)
    pos = pos_f.reshape(4, 32, NUM, 4)
    return tw, pos
